# baseline (speedup 1.0000x reference)
import functools

import jax
import jax.numpy as jnp
from jax import lax
from jax.experimental import pallas as pl
from jax.experimental.pallas import tpu as pltpu
from jax.experimental.pallas import tpu_sc as plsc

N, P1, P2, D = 8, 2048, 2048, 3
SQ = 512
NQ = P2 // SQ

_BIG = float("inf")


def _argmin_kernel(x_ref, yt_ref, yidx_ref, xidx_ref):
    xb = x_ref[0]
    ytf = yt_ref[0]

    x2c = jnp.sum(xb * xb, axis=1, keepdims=True)
    xm2 = xb * (-2.0)
    qiof = lax.broadcasted_iota(jnp.int32, (P1, SQ), 1).astype(jnp.float32)
    piof = lax.broadcasted_iota(jnp.int32, (P1, SQ), 0).astype(jnp.float32)

    tmins, tidxs = [], []
    for iq in range(NQ):
        ytb = ytf[:, iq * SQ:(iq + 1) * SQ]
        y2r = jnp.sum(ytb * ytb, axis=0, keepdims=True)
        ab2 = lax.dot_general(xm2, ytb,
                              dimension_numbers=(((1,), (0,)), ((), ())),
                              preferred_element_type=jnp.float32,
                              precision=lax.Precision.DEFAULT)
        d2 = (x2c + y2r) + ab2

        cmin = jnp.min(d2, axis=0, keepdims=True)
        cidx = jnp.min(jnp.where(d2 <= cmin, piof, 65536.0),
                       axis=0, keepdims=True)
        yidx_ref[0, :, iq * SQ:(iq + 1) * SQ] = cidx.astype(jnp.int32)

        tmin = jnp.min(d2, axis=1, keepdims=True)
        tidx = jnp.min(jnp.where(d2 <= tmin, qiof, 65536.0),
                       axis=1, keepdims=True) + float(iq * SQ)
        tmins.append(tmin)
        tidxs.append(tidx)

    m, mi = tmins[0], tidxs[0]
    for k in range(1, NQ):
        upd = tmins[k] < m
        m = jnp.where(upd, tmins[k], m)
        mi = jnp.where(upd, tidxs[k], mi)
    xidx_ref[0] = mi.astype(jnp.int32).reshape(1, P1)


def _argmin_call(x, yt):
    return pl.pallas_call(
        _argmin_kernel,
        grid=(N,),
        in_specs=[
            pl.BlockSpec((1, P1, D), lambda n: (n, 0, 0)),
            pl.BlockSpec((1, D, P2), lambda n: (n, 0, 0)),
        ],
        out_specs=[
            pl.BlockSpec((1, 1, P2), lambda n: (n, 0, 0)),
            pl.BlockSpec((1, 1, P1), lambda n: (n, 0, 0)),
        ],
        out_shape=[
            jax.ShapeDtypeStruct((N, 1, P2), jnp.int32),
            jax.ShapeDtypeStruct((N, 1, P1), jnp.int32),
        ],
    )(x, yt)


_PTS = 1024
_STEPS = _PTS // 16


def _sqrt16(x):
    i = plsc.bitcast(x, jnp.int32)
    i = jnp.int32(0x5F3759DF) - (i >> 1)
    yv = plsc.bitcast(i, jnp.float32)
    xh = 0.5 * x
    for _ in range(3):
        yv = yv * (1.5 - xh * yv * yv)
    return x * yv


def _sc_finish_kernel(x0, x1, x2, nx0, nx1, nx2,
                      y0, y1, y2, ny0, ny1, ny2,
                      xidx, yidx,
                      x2y_out, y2x_out,
                      t0, t1, t2, tn0, tn1, tn2,
                      q0r, q1r, q2r, idxr, outr):
    c = lax.axis_index("c")
    s = lax.axis_index("s")
    n = s // 2
    p0 = (s % 2) * _PTS
    nb = n * 2048
    base = nb + p0

    @pl.when(c == 0)
    def _stage_xdir():
        pltpu.sync_copy(y0.at[pl.ds(nb, 2048)], t0)
        pltpu.sync_copy(y1.at[pl.ds(nb, 2048)], t1)
        pltpu.sync_copy(y2.at[pl.ds(nb, 2048)], t2)
        pltpu.sync_copy(ny0.at[pl.ds(nb, 2048)], tn0)
        pltpu.sync_copy(ny1.at[pl.ds(nb, 2048)], tn1)
        pltpu.sync_copy(ny2.at[pl.ds(nb, 2048)], tn2)
        pltpu.sync_copy(x0.at[pl.ds(base, _PTS)], q0r)
        pltpu.sync_copy(x1.at[pl.ds(base, _PTS)], q1r)
        pltpu.sync_copy(x2.at[pl.ds(base, _PTS)], q2r)
        pltpu.sync_copy(xidx.at[pl.ds(base, _PTS)], idxr)

    @pl.when(c == 1)
    def _stage_ydir():
        pltpu.sync_copy(x0.at[pl.ds(nb, 2048)], t0)
        pltpu.sync_copy(x1.at[pl.ds(nb, 2048)], t1)
        pltpu.sync_copy(x2.at[pl.ds(nb, 2048)], t2)
        pltpu.sync_copy(nx0.at[pl.ds(nb, 2048)], tn0)
        pltpu.sync_copy(nx1.at[pl.ds(nb, 2048)], tn1)
        pltpu.sync_copy(nx2.at[pl.ds(nb, 2048)], tn2)
        pltpu.sync_copy(y0.at[pl.ds(base, _PTS)], q0r)
        pltpu.sync_copy(y1.at[pl.ds(base, _PTS)], q1r)
        pltpu.sync_copy(y2.at[pl.ds(base, _PTS)], q2r)
        pltpu.sync_copy(yidx.at[pl.ds(base, _PTS)], idxr)

    def body(i, _):
        o = i * 16
        idx16 = idxr[pl.ds(o, 16)]
        gx = plsc.load_gather(t0, [idx16])
        gy = plsc.load_gather(t1, [idx16])
        gz = plsc.load_gather(t2, [idx16])
        nx = plsc.load_gather(tn0, [idx16])
        ny = plsc.load_gather(tn1, [idx16])
        nz = plsc.load_gather(tn2, [idx16])
        dx = q0r[pl.ds(o, 16)] - gx
        dy = q1r[pl.ds(o, 16)] - gy
        dz = q2r[pl.ds(o, 16)] - gz
        d2e = dx * dx + dy * dy + dz * dz
        sd = nx * dx + ny * dy + nz * dz
        outr[pl.ds(o, 16)] = _sqrt16(d2e) * jnp.sign(sd)
        return _

    lax.fori_loop(0, _STEPS, body, None)

    @pl.when(c == 0)
    def _out_xdir():
        pltpu.sync_copy(outr, x2y_out.at[pl.ds(base, _PTS)])

    @pl.when(c == 1)
    def _out_ydir():
        pltpu.sync_copy(outr, y2x_out.at[pl.ds(base, _PTS)])


def _sc_finish_call(xc, nxc, yc, nyc, xidx_flat, yidx_flat):
    mesh = plsc.VectorSubcoreMesh(core_axis_name="c", subcore_axis_name="s",
                                  num_cores=2)
    kern = pl.kernel(
        _sc_finish_kernel,
        mesh=mesh,
        compiler_params=pltpu.CompilerParams(needs_layout_passes=False),
        out_type=[
            jax.ShapeDtypeStruct((N * P1,), jnp.float32),
            jax.ShapeDtypeStruct((N * P2,), jnp.float32),
        ],
        scratch_types=[
            pltpu.VMEM((P2,), jnp.float32),
            pltpu.VMEM((P2,), jnp.float32),
            pltpu.VMEM((P2,), jnp.float32),
            pltpu.VMEM((P2,), jnp.float32),
            pltpu.VMEM((P2,), jnp.float32),
            pltpu.VMEM((P2,), jnp.float32),
            pltpu.VMEM((_PTS,), jnp.float32),
            pltpu.VMEM((_PTS,), jnp.float32),
            pltpu.VMEM((_PTS,), jnp.float32),
            pltpu.VMEM((_PTS,), jnp.int32),
            pltpu.VMEM((_PTS,), jnp.float32),
        ],
    )
    return kern(*xc, *nxc, *yc, *nyc, xidx_flat, yidx_flat)


def kernel(x, y, x_normals, y_normals):
    yt = jnp.swapaxes(y, 1, 2)

    yidx3, xidx3 = _argmin_call(x, yt)
    yidx = yidx3.reshape(N, P2)
    xidx = xidx3.reshape(N, P1)

    def comps(a):
        return tuple(a[:, :, i].reshape(-1) for i in range(3))

    x2y_flat, y2x_flat = _sc_finish_call(
        comps(x), comps(x_normals), comps(y), comps(y_normals),
        xidx.reshape(-1), yidx.reshape(-1))

    return (y2x_flat.reshape(N, P2), x2y_flat.reshape(N, P1), yidx, xidx)


# speedup vs baseline: 1.0118x; 1.0118x over previous
import functools

import jax
import jax.numpy as jnp
from jax import lax
from jax.experimental import pallas as pl
from jax.experimental.pallas import tpu as pltpu
from jax.experimental.pallas import tpu_sc as plsc

N, P1, P2, D = 8, 2048, 2048, 3
SQ = 512
NQ = P2 // SQ

_BIG = float("inf")


def _argmin_kernel(x_ref, yt_ref, yidx_ref, xidx_ref):
    xb = x_ref[0]
    ytf = yt_ref[0]

    x2c = jnp.sum(xb * xb, axis=1, keepdims=True)
    xm2 = xb * (-2.0)
    qiof = lax.broadcasted_iota(jnp.int32, (P1, SQ), 1).astype(jnp.float32)
    piof = lax.broadcasted_iota(jnp.int32, (P1, SQ), 0).astype(jnp.float32)

    tmins, tidxs = [], []
    for iq in range(NQ):
        ytb = ytf[:, iq * SQ:(iq + 1) * SQ]
        y2r = jnp.sum(ytb * ytb, axis=0, keepdims=True)
        ab2 = lax.dot_general(xm2, ytb,
                              dimension_numbers=(((1,), (0,)), ((), ())),
                              preferred_element_type=jnp.float32,
                              precision=lax.Precision.DEFAULT)
        d2 = (x2c + y2r) + ab2

        cmin = jnp.min(d2, axis=0, keepdims=True)
        cidx = jnp.min(jnp.where(d2 <= cmin, piof, 65536.0),
                       axis=0, keepdims=True)
        yidx_ref[0, :, iq * SQ:(iq + 1) * SQ] = cidx.astype(jnp.int32)

        tmin = jnp.min(d2, axis=1, keepdims=True)
        tidx = jnp.min(jnp.where(d2 <= tmin, qiof, 65536.0),
                       axis=1, keepdims=True) + float(iq * SQ)
        tmins.append(tmin)
        tidxs.append(tidx)

    m, mi = tmins[0], tidxs[0]
    for k in range(1, NQ):
        upd = tmins[k] < m
        m = jnp.where(upd, tmins[k], m)
        mi = jnp.where(upd, tidxs[k], mi)
    xidx_ref[0] = mi.astype(jnp.int32).reshape(1, P1)


def _argmin_call(x, yt):
    return pl.pallas_call(
        _argmin_kernel,
        grid=(N,),
        in_specs=[
            pl.BlockSpec((1, P1, D), lambda n: (n, 0, 0)),
            pl.BlockSpec((1, D, P2), lambda n: (n, 0, 0)),
        ],
        out_specs=[
            pl.BlockSpec((1, 1, P2), lambda n: (n, 0, 0)),
            pl.BlockSpec((1, 1, P1), lambda n: (n, 0, 0)),
        ],
        out_shape=[
            jax.ShapeDtypeStruct((N, 1, P2), jnp.int32),
            jax.ShapeDtypeStruct((N, 1, P1), jnp.int32),
        ],
    )(x, yt)


_PTS = 1024
_STEPS = _PTS // 16


def _sqrt16(x):
    i = plsc.bitcast(x, jnp.int32)
    i = jnp.int32(0x5F3759DF) - (i >> 1)
    yv = plsc.bitcast(i, jnp.float32)
    xh = 0.5 * x
    for _ in range(3):
        yv = yv * (1.5 - xh * yv * yv)
    return x * yv


def _sc_finish_kernel(x0, x1, x2, nx0, nx1, nx2,
                      y0, y1, y2, ny0, ny1, ny2,
                      xidx, yidx,
                      x2y_out, y2x_out,
                      t0, t1, t2, tn0, tn1, tn2,
                      q0r, q1r, q2r, idxr, outr):
    c = lax.axis_index("c")
    s = lax.axis_index("s")
    n = s // 2
    p0 = (s % 2) * _PTS
    nb = n * 2048
    base = nb + p0

    @pl.when(c == 0)
    def _stage_xdir():
        pltpu.sync_copy(y0.at[pl.ds(nb, 2048)], t0)
        pltpu.sync_copy(y1.at[pl.ds(nb, 2048)], t1)
        pltpu.sync_copy(y2.at[pl.ds(nb, 2048)], t2)
        pltpu.sync_copy(ny0.at[pl.ds(nb, 2048)], tn0)
        pltpu.sync_copy(ny1.at[pl.ds(nb, 2048)], tn1)
        pltpu.sync_copy(ny2.at[pl.ds(nb, 2048)], tn2)
        pltpu.sync_copy(x0.at[pl.ds(base, _PTS)], q0r)
        pltpu.sync_copy(x1.at[pl.ds(base, _PTS)], q1r)
        pltpu.sync_copy(x2.at[pl.ds(base, _PTS)], q2r)
        pltpu.sync_copy(xidx.at[pl.ds(base, _PTS)], idxr)

    @pl.when(c == 1)
    def _stage_ydir():
        pltpu.sync_copy(x0.at[pl.ds(nb, 2048)], t0)
        pltpu.sync_copy(x1.at[pl.ds(nb, 2048)], t1)
        pltpu.sync_copy(x2.at[pl.ds(nb, 2048)], t2)
        pltpu.sync_copy(nx0.at[pl.ds(nb, 2048)], tn0)
        pltpu.sync_copy(nx1.at[pl.ds(nb, 2048)], tn1)
        pltpu.sync_copy(nx2.at[pl.ds(nb, 2048)], tn2)
        pltpu.sync_copy(y0.at[pl.ds(base, _PTS)], q0r)
        pltpu.sync_copy(y1.at[pl.ds(base, _PTS)], q1r)
        pltpu.sync_copy(y2.at[pl.ds(base, _PTS)], q2r)
        pltpu.sync_copy(yidx.at[pl.ds(base, _PTS)], idxr)

    def body(i, _):
        o = i * 16
        idx16 = idxr[pl.ds(o, 16)]
        gx = plsc.load_gather(t0, [idx16])
        gy = plsc.load_gather(t1, [idx16])
        gz = plsc.load_gather(t2, [idx16])
        nx = plsc.load_gather(tn0, [idx16])
        ny = plsc.load_gather(tn1, [idx16])
        nz = plsc.load_gather(tn2, [idx16])
        dx = q0r[pl.ds(o, 16)] - gx
        dy = q1r[pl.ds(o, 16)] - gy
        dz = q2r[pl.ds(o, 16)] - gz
        d2e = dx * dx + dy * dy + dz * dz
        sd = nx * dx + ny * dy + nz * dz
        outr[pl.ds(o, 16)] = _sqrt16(d2e) * jnp.sign(sd)
        return _

    lax.fori_loop(0, _STEPS, body, None)

    @pl.when(c == 0)
    def _out_xdir():
        pltpu.sync_copy(outr, x2y_out.at[pl.ds(base, _PTS)])

    @pl.when(c == 1)
    def _out_ydir():
        pltpu.sync_copy(outr, y2x_out.at[pl.ds(base, _PTS)])


def _sc_finish_call(xc, nxc, yc, nyc, xidx_flat, yidx_flat):
    mesh = plsc.VectorSubcoreMesh(core_axis_name="c", subcore_axis_name="s",
                                  num_cores=2)
    kern = pl.kernel(
        _sc_finish_kernel,
        mesh=mesh,
        compiler_params=pltpu.CompilerParams(needs_layout_passes=False),
        out_type=[
            jax.ShapeDtypeStruct((N * P1,), jnp.float32),
            jax.ShapeDtypeStruct((N * P2,), jnp.float32),
        ],
        scratch_types=[
            pltpu.VMEM((P2,), jnp.float32),
            pltpu.VMEM((P2,), jnp.float32),
            pltpu.VMEM((P2,), jnp.float32),
            pltpu.VMEM((P2,), jnp.float32),
            pltpu.VMEM((P2,), jnp.float32),
            pltpu.VMEM((P2,), jnp.float32),
            pltpu.VMEM((_PTS,), jnp.float32),
            pltpu.VMEM((_PTS,), jnp.float32),
            pltpu.VMEM((_PTS,), jnp.float32),
            pltpu.VMEM((_PTS,), jnp.int32),
            pltpu.VMEM((_PTS,), jnp.float32),
        ],
    )
    return kern(*xc, *nxc, *yc, *nyc, xidx_flat, yidx_flat)


def kernel(x, y, x_normals, y_normals):
    yt = jnp.swapaxes(y, 1, 2)

    yidx3, xidx3 = _argmin_call(x, yt)
    yidx = yidx3.reshape(N, P2)
    xidx = xidx3.reshape(N, P1)

    def comps(a):
        at = jnp.swapaxes(a, 1, 2)
        return tuple(at[:, i, :].reshape(-1) for i in range(3))

    x2y_flat, y2x_flat = _sc_finish_call(
        comps(x), comps(x_normals), comps(y), comps(y_normals),
        xidx.reshape(-1), yidx.reshape(-1))

    return (y2x_flat.reshape(N, P2), x2y_flat.reshape(N, P1), yidx, xidx)
